# baseline (speedup 1.0000x reference)
import jax
import jax.numpy as jnp
from jax import lax
from jax.experimental import pallas as pl
from jax.experimental.pallas import tpu as pltpu
from jax.experimental.pallas import tpu_sc as plsc

N = 50000
D = 128
NP = 50176
NCH = 8
CW = 16
E = 600000
EP = 614400
NS = 16
NC = 2
EPT = EP // NS
B = 512
NIT = EPT // B
SB = 480
SNIT = EPT // SB
NR = 4
RPT = NP // NS
ZR = 448
BN = 1792
GN = NP // BN

_mesh = plsc.VectorSubcoreMesh(core_axis_name="c", subcore_axis_name="s")


def _deg_body(src_hbm, dst_hbm, out_hbm, idx_v, ones_v, zf_v, acc_sh):
    cid = lax.axis_index("c")
    sid = lax.axis_index("s")
    for k in range(B // 16):
        ones_v[pl.ds(k * 16, 16)] = jnp.full((16,), 1.0, jnp.float32)
    for k in range(ZR * CW // 16):
        zf_v[pl.ds(k * 16, 16)] = jnp.zeros((16,), jnp.float32)
    for z in range(RPT // ZR):
        pltpu.sync_copy(zf_v.at[pl.ds(0, ZR)], acc_sh.at[pl.ds(sid * RPT + z * ZR, ZR)])
    plsc.subcore_barrier()

    def t_body(t, carry):
        base = sid * EPT + t * B

        @pl.when(cid == 0)
        def _():
            pltpu.sync_copy(src_hbm.at[pl.ds(base, B)], idx_v)

        @pl.when(cid == 1)
        def _():
            pltpu.sync_copy(dst_hbm.at[pl.ds(base, B)], idx_v)

        pltpu.sync_copy(ones_v, acc_sh.at[idx_v], add=True)
        return carry

    lax.fori_loop(0, NIT, t_body, 0)
    plsc.subcore_barrier()
    wb = sid * RPT
    for z in range(RPT // ZR):
        pltpu.sync_copy(acc_sh.at[pl.ds(wb + z * ZR, ZR)], zf_v.at[pl.ds(0, ZR)])
        pltpu.sync_copy(zf_v.at[pl.ds(0, ZR)],
                        out_hbm.at[pl.ds(cid * NP + wb + z * ZR, ZR)])


_deg_call = pl.kernel(
    _deg_body,
    out_type=jax.ShapeDtypeStruct((NC * NP,), jnp.float32),
    mesh=_mesh,
    scratch_types=[
        pltpu.VMEM((B,), jnp.int32),
        pltpu.VMEM((B,), jnp.float32),
        pltpu.VMEM((ZR * CW,), jnp.float32),
        pltpu.VMEM_SHARED((NP,), jnp.float32),
    ],
)


def _seg_body(table_hbm, gi_hbm, si_hbm, out_hbm,
              gi0, si0, r0, sg0, ss0, gi1, si1, r1, sg1, ss1,
              gi2, si2, r2, sg2, ss2, gi3, si3, r3, sg3, ss3, acc_sh):
    cid = lax.axis_index("c")
    sid = lax.axis_index("s")
    gi = (gi0, gi1, gi2, gi3)
    si = (si0, si1, si2, si3)
    rows = (r0, r1, r2, r3)
    sem_g = (sg0, sg1, sg2, sg3)
    sem_s = (ss0, ss1, ss2, ss3)

    def load_adjust_fire(t, s, off):
        base = sid * EPT + t * SB
        pltpu.sync_copy(gi_hbm.at[pl.ds(base, SB)], gi[s])
        pltpu.sync_copy(si_hbm.at[pl.ds(base, SB)], si[s])
        for k in range(SB // 16):
            gi[s][pl.ds(k * 16, 16)] = gi[s][pl.ds(k * 16, 16)] + off
        pltpu.async_copy(table_hbm.at[gi[s]], rows[s], sem_g[s])

    for j in range(NCH // NC):
        off = (cid * (NCH // NC) + j) * NP
        for r in range(SB):
            for h in range(CW // 16):
                r0[r, pl.ds(h * 16, 16)] = jnp.zeros((16,), jnp.float32)
        for z in range(RPT // SB):
            pltpu.sync_copy(r0.at[pl.ds(0, SB)],
                            acc_sh.at[pl.ds(sid * RPT + z * SB, SB)])
        pltpu.sync_copy(r0.at[pl.ds(0, RPT % SB)],
                        acc_sh.at[pl.ds(sid * RPT + (RPT // SB) * SB, RPT % SB)])
        plsc.subcore_barrier()

        load_adjust_fire(0, 0, off)
        load_adjust_fire(1, 1, off)

        def u_body(u, carry):
            for b in range(NR):
                t = NR * u + b
                s = b
                sp = (b + 2) % NR
                pltpu.make_async_copy(table_hbm.at[gi[s]], rows[s],
                                      sem_g[s]).wait()
                pltpu.async_copy(rows[s], acc_sh.at[si[s]], sem_s[s], add=True)

                @pl.when(t >= 2)
                def _():
                    pltpu.make_async_copy(rows[sp], acc_sh.at[si[sp]],
                                          sem_s[sp]).wait()

                @pl.when(t + 2 < SNIT)
                def _():
                    load_adjust_fire(t + 2, sp, off)
            return carry

        lax.fori_loop(0, SNIT // NR, u_body, 0)
        for s in ((SNIT - 2) % NR, (SNIT - 1) % NR):
            pltpu.make_async_copy(rows[s], acc_sh.at[si[s]], sem_s[s]).wait()
        plsc.subcore_barrier()
        wb = sid * RPT
        for z in range(RPT // SB):
            pltpu.sync_copy(acc_sh.at[pl.ds(wb + z * SB, SB)], r0)
            pltpu.sync_copy(r0, out_hbm.at[pl.ds(off + wb + z * SB, SB)])
        pltpu.sync_copy(acc_sh.at[pl.ds(wb + (RPT // SB) * SB, RPT % SB)],
                        r0.at[pl.ds(0, RPT % SB)])
        pltpu.sync_copy(r0.at[pl.ds(0, RPT % SB)],
                        out_hbm.at[pl.ds(off + wb + (RPT // SB) * SB, RPT % SB)])
        plsc.subcore_barrier()


_seg_call = pl.kernel(
    _seg_body,
    out_type=jax.ShapeDtypeStruct((NCH * NP, CW), jnp.float32),
    mesh=_mesh,
    compiler_params=pltpu.CompilerParams(use_tc_tiling_on_sc=False),
    scratch_types=[
        pltpu.VMEM((SB,), jnp.int32),
        pltpu.VMEM((SB,), jnp.int32),
        pltpu.VMEM((SB, CW), jnp.float32),
        pltpu.SemaphoreType.DMA,
        pltpu.SemaphoreType.DMA,
    ] * NR + [
        pltpu.VMEM_SHARED((NP, CW), jnp.float32),
    ],
)


def _scales_kernel(deg_ref, a_ref, b_ref, a2_ref, b2_ref):
    i = pl.program_id(0)
    rows = lax.broadcasted_iota(jnp.int32, (BN, 1), 0) + i * BN
    valid = rows < N
    a = jnp.where(valid, lax.rsqrt(jnp.maximum(deg_ref[0], 1.0)), 0.0)
    b = jnp.where(valid, lax.rsqrt(jnp.maximum(deg_ref[1], 1.0)), 0.0)
    a_ref[...] = a
    b_ref[...] = b
    a2_ref[...] = a * a
    b2_ref[...] = b * b


def _scales_call(deg):
    s = jax.ShapeDtypeStruct((NP, 1), jnp.float32)
    return pl.pallas_call(
        _scales_kernel,
        grid=(GN,),
        in_specs=[pl.BlockSpec((NC, BN, 1), lambda i: (0, i, 0))],
        out_specs=[pl.BlockSpec((BN, 1), lambda i: (i, 0))] * 4,
        out_shape=[s, s, s, s],
    )(deg.reshape(NC, NP, 1))


def _init_user_kernel(x_ref, a_ref, out_ref):
    i = pl.program_id(0)
    rows = lax.broadcasted_iota(jnp.int32, (BN, 1), 0) + i * BN
    valid = rows < N
    h = jnp.where(valid, a_ref[...] * x_ref[...], 0.0)
    for c in range(NCH):
        out_ref[c] = h[:, c * CW:(c + 1) * CW]


def _init_user_call(user_emb, a):
    return pl.pallas_call(
        _init_user_kernel,
        grid=(GN,),
        in_specs=[
            pl.BlockSpec((BN, D), lambda i: (i, 0)),
            pl.BlockSpec((BN, 1), lambda i: (i, 0)),
        ],
        out_specs=pl.BlockSpec((NCH, BN, CW), lambda i: (0, i, 0)),
        out_shape=jax.ShapeDtypeStruct((NCH, NP, CW), jnp.float32),
    )(user_emb, a)


def _init_item_kernel(x_ref, w_ref, bias_ref, b_ref, out_ref):
    i = pl.program_id(0)
    rows = lax.broadcasted_iota(jnp.int32, (BN, 1), 0) + i * BN
    valid = rows < N
    h = lax.dot_general(x_ref[...], w_ref[...], (((1,), (1,)), ((), ())),
                        preferred_element_type=jnp.float32) + bias_ref[...]
    h = jnp.where(valid, b_ref[...] * h, 0.0)
    for c in range(NCH):
        out_ref[c] = h[:, c * CW:(c + 1) * CW]


def _init_item_call(item_emb, fc_w, fc_b, b):
    return pl.pallas_call(
        _init_item_kernel,
        grid=(GN,),
        in_specs=[
            pl.BlockSpec((BN, D), lambda i: (i, 0)),
            pl.BlockSpec((D, D), lambda i: (0, 0)),
            pl.BlockSpec((1, D), lambda i: (0, 0)),
            pl.BlockSpec((BN, 1), lambda i: (i, 0)),
        ],
        out_specs=pl.BlockSpec((NCH, BN, CW), lambda i: (0, i, 0)),
        out_shape=jax.ShapeDtypeStruct((NCH, NP, CW), jnp.float32),
    )(item_emb, fc_w, fc_b.reshape(1, D), b)


def _scale_kernel(s_ref, sc_ref, out_ref):
    i = pl.program_id(1)
    rows = lax.broadcasted_iota(jnp.int32, (1, BN, 1), 1) + i * BN
    valid = rows < N
    out_ref[...] = jnp.where(valid, sc_ref[...] * s_ref[...], 0.0)


def _scale_call(s, sc):
    return pl.pallas_call(
        _scale_kernel,
        grid=(NCH, GN),
        in_specs=[
            pl.BlockSpec((1, BN, CW), lambda c, i: (c, i, 0)),
            pl.BlockSpec((1, BN, 1), lambda c, i: (0, i, 0)),
        ],
        out_specs=pl.BlockSpec((1, BN, CW), lambda c, i: (c, i, 0)),
        out_shape=jax.ShapeDtypeStruct((NCH, NP, CW), jnp.float32),
    )(s, sc.reshape(1, NP, 1))


def _final_kernel(x_ref, sc_ref, s1_ref, s2_ref, out_ref):
    s1 = jnp.concatenate([s1_ref[c] for c in range(NCH)], axis=1)
    s2 = jnp.concatenate([s2_ref[c] for c in range(NCH)], axis=1)
    out_ref[...] = (x_ref[...] + sc_ref[...] * (s1 + s2)) * (1.0 / 3.0)


def _final_call(x, sc, s1, s2):
    return pl.pallas_call(
        _final_kernel,
        grid=(GN,),
        in_specs=[
            pl.BlockSpec((BN, D), lambda i: (i, 0)),
            pl.BlockSpec((BN, 1), lambda i: (i, 0)),
            pl.BlockSpec((NCH, BN, CW), lambda i: (0, i, 0)),
            pl.BlockSpec((NCH, BN, CW), lambda i: (0, i, 0)),
        ],
        out_specs=pl.BlockSpec((BN, D), lambda i: (i, 0)),
        out_shape=jax.ShapeDtypeStruct((N, D), jnp.float32),
    )(x, sc, s1, s2)


@jax.jit
def kernel(user_emb, item_emb, fc_w, fc_b, edge_src, edge_dst):
    pad = jnp.arange(EP - E, dtype=jnp.int32) % (NP - N) + N
    src_p = jnp.concatenate([edge_src, pad])
    dst_p = jnp.concatenate([edge_dst, pad])

    deg = _deg_call(src_p, dst_p)
    a, b, a2, b2 = _scales_call(deg)

    xu0 = _init_user_call(user_emb, a)
    xi0 = _init_item_call(item_emb, fc_w, fc_b, b)

    si1 = _seg_call(xu0.reshape(NCH * NP, CW), src_p, dst_p)
    su1 = _seg_call(xi0.reshape(NCH * NP, CW), dst_p, src_p)

    xi1 = _scale_call(si1.reshape(NCH, NP, CW), b2)
    xu1 = _scale_call(su1.reshape(NCH, NP, CW), a2)

    su2 = _seg_call(xi1.reshape(NCH * NP, CW), dst_p, src_p)
    si2 = _seg_call(xu1.reshape(NCH * NP, CW), src_p, dst_p)

    user_out = _final_call(user_emb, a, su1.reshape(NCH, NP, CW),
                           su2.reshape(NCH, NP, CW))
    item_out = _final_call(item_emb, b, si1.reshape(NCH, NP, CW),
                           si2.reshape(NCH, NP, CW))
    return user_out, item_out


# speedup vs baseline: 1.4359x; 1.4359x over previous
import jax
import jax.numpy as jnp
from jax import lax
from jax.experimental import pallas as pl
from jax.experimental.pallas import tpu as pltpu
from jax.experimental.pallas import tpu_sc as plsc

N = 50000
D = 128
NP = 50176
NCH = 4
CW = 32
E = 600000
EP = 614400
NS = 16
NC = 2
EPT = EP // NS
B = 512
NIT = EPT // B
SB = 128
SNIT = EPT // SB
NR = 5
RPT = NP // NS
ZR = 448
BN = 1792
GN = NP // BN

_mesh = plsc.VectorSubcoreMesh(core_axis_name="c", subcore_axis_name="s")


def _deg_body(src_hbm, dst_hbm, out_hbm, idx_v, ones_v, zf_v, acc_sh):
    cid = lax.axis_index("c")
    sid = lax.axis_index("s")
    for k in range(B // 16):
        ones_v[pl.ds(k * 16, 16)] = jnp.full((16,), 1.0, jnp.float32)
    for k in range(ZR * CW // 16):
        zf_v[pl.ds(k * 16, 16)] = jnp.zeros((16,), jnp.float32)
    for z in range(RPT // ZR):
        pltpu.sync_copy(zf_v.at[pl.ds(0, ZR)], acc_sh.at[pl.ds(sid * RPT + z * ZR, ZR)])
    plsc.subcore_barrier()

    def t_body(t, carry):
        base = sid * EPT + t * B

        @pl.when(cid == 0)
        def _():
            pltpu.sync_copy(src_hbm.at[pl.ds(base, B)], idx_v)

        @pl.when(cid == 1)
        def _():
            pltpu.sync_copy(dst_hbm.at[pl.ds(base, B)], idx_v)

        pltpu.sync_copy(ones_v, acc_sh.at[idx_v], add=True)
        return carry

    lax.fori_loop(0, NIT, t_body, 0)
    plsc.subcore_barrier()
    wb = sid * RPT
    for z in range(RPT // ZR):
        pltpu.sync_copy(acc_sh.at[pl.ds(wb + z * ZR, ZR)], zf_v.at[pl.ds(0, ZR)])
        pltpu.sync_copy(zf_v.at[pl.ds(0, ZR)],
                        out_hbm.at[pl.ds(cid * NP + wb + z * ZR, ZR)])


_deg_call = pl.kernel(
    _deg_body,
    out_type=jax.ShapeDtypeStruct((NC * NP,), jnp.float32),
    mesh=_mesh,
    scratch_types=[
        pltpu.VMEM((B,), jnp.int32),
        pltpu.VMEM((B,), jnp.float32),
        pltpu.VMEM((ZR * CW,), jnp.float32),
        pltpu.VMEM_SHARED((NP,), jnp.float32),
    ],
)


def _seg_body(table_hbm, gi_hbm, si_hbm, out_hbm, *scratch):
    acc_sh = scratch[-1]
    gi = tuple(scratch[6 * s + 0] for s in range(NR))
    si = tuple(scratch[6 * s + 1] for s in range(NR))
    rows = tuple(scratch[6 * s + 2] for s in range(NR))
    sem_g = tuple(scratch[6 * s + 3] for s in range(NR))
    sem_s = tuple(scratch[6 * s + 4] for s in range(NR))
    sem_i = tuple(scratch[6 * s + 5] for s in range(NR))
    cid = lax.axis_index("c")
    sid = lax.axis_index("s")
    r0 = rows[0]

    def fire_idx(t, s):
        base = sid * EPT + t * SB
        pltpu.async_copy(gi_hbm.at[pl.ds(base, SB)], gi[s], sem_i[s])
        pltpu.async_copy(si_hbm.at[pl.ds(base, SB)], si[s], sem_i[s])

    def wait_idx_adjust_fire_gather(t, s, off):
        base = sid * EPT + t * SB
        pltpu.make_async_copy(gi_hbm.at[pl.ds(base, SB)], gi[s],
                              sem_i[s]).wait()
        pltpu.make_async_copy(si_hbm.at[pl.ds(base, SB)], si[s],
                              sem_i[s]).wait()
        for k in range(SB // 16):
            gi[s][pl.ds(k * 16, 16)] = gi[s][pl.ds(k * 16, 16)] + off
        pltpu.async_copy(table_hbm.at[gi[s]], rows[s], sem_g[s])

    for j in range(NCH // NC):
        off = (cid * (NCH // NC) + j) * NP
        for r in range(SB):
            for h in range(CW // 16):
                r0[r, pl.ds(h * 16, 16)] = jnp.zeros((16,), jnp.float32)
        for z in range(RPT // SB):
            pltpu.sync_copy(r0.at[pl.ds(0, SB)],
                            acc_sh.at[pl.ds(sid * RPT + z * SB, SB)])
        pltpu.sync_copy(r0.at[pl.ds(0, RPT % SB)],
                        acc_sh.at[pl.ds(sid * RPT + (RPT // SB) * SB, RPT % SB)])
        plsc.subcore_barrier()

        for s in (0, 1):
            base = sid * EPT + s * SB
            pltpu.sync_copy(gi_hbm.at[pl.ds(base, SB)], gi[s])
            pltpu.sync_copy(si_hbm.at[pl.ds(base, SB)], si[s])
            for k in range(SB // 16):
                gi[s][pl.ds(k * 16, 16)] = gi[s][pl.ds(k * 16, 16)] + off
            pltpu.async_copy(table_hbm.at[gi[s]], rows[s], sem_g[s])
        fire_idx(2, 2)

        def u_body(u, carry):
            for b in range(NR):
                t = NR * u + b
                s = b
                pltpu.make_async_copy(table_hbm.at[gi[s]], rows[s],
                                      sem_g[s]).wait()
                pltpu.async_copy(rows[s], acc_sh.at[si[s]], sem_s[s], add=True)

                @pl.when(t >= 2)
                def _():
                    s2 = (b + 3) % NR
                    pltpu.make_async_copy(rows[s2], acc_sh.at[si[s2]],
                                          sem_s[s2]).wait()

                @pl.when(t + 3 < SNIT)
                def _():
                    fire_idx(t + 3, (b + 3) % NR)

                @pl.when(t + 2 < SNIT)
                def _():
                    wait_idx_adjust_fire_gather(t + 2, (b + 2) % NR, off)
            return carry

        lax.fori_loop(0, SNIT // NR, u_body, 0)
        for st in (SNIT - 2, SNIT - 1):
            s = st % NR
            pltpu.make_async_copy(rows[s], acc_sh.at[si[s]], sem_s[s]).wait()
        plsc.subcore_barrier()
        wb = sid * RPT
        for z in range(RPT // SB):
            pltpu.sync_copy(acc_sh.at[pl.ds(wb + z * SB, SB)], r0)
            pltpu.sync_copy(r0, out_hbm.at[pl.ds(off + wb + z * SB, SB)])
        pltpu.sync_copy(acc_sh.at[pl.ds(wb + (RPT // SB) * SB, RPT % SB)],
                        r0.at[pl.ds(0, RPT % SB)])
        pltpu.sync_copy(r0.at[pl.ds(0, RPT % SB)],
                        out_hbm.at[pl.ds(off + wb + (RPT // SB) * SB, RPT % SB)])
        plsc.subcore_barrier()


_seg_call = pl.kernel(
    _seg_body,
    out_type=jax.ShapeDtypeStruct((NCH * NP, CW), jnp.float32),
    mesh=_mesh,
    compiler_params=pltpu.CompilerParams(use_tc_tiling_on_sc=False),
    scratch_types=[
        pltpu.VMEM((SB,), jnp.int32),
        pltpu.VMEM((SB,), jnp.int32),
        pltpu.VMEM((SB, CW), jnp.float32),
        pltpu.SemaphoreType.DMA,
        pltpu.SemaphoreType.DMA,
        pltpu.SemaphoreType.DMA,
    ] * NR + [
        pltpu.VMEM_SHARED((NP, CW), jnp.float32),
    ],
)


def _scales_kernel(deg_ref, a_ref, b_ref, a2_ref, b2_ref):
    i = pl.program_id(0)
    rows = lax.broadcasted_iota(jnp.int32, (BN, 1), 0) + i * BN
    valid = rows < N
    a = jnp.where(valid, lax.rsqrt(jnp.maximum(deg_ref[0], 1.0)), 0.0)
    b = jnp.where(valid, lax.rsqrt(jnp.maximum(deg_ref[1], 1.0)), 0.0)
    a_ref[...] = a
    b_ref[...] = b
    a2_ref[...] = a * a
    b2_ref[...] = b * b


def _scales_call(deg):
    s = jax.ShapeDtypeStruct((NP, 1), jnp.float32)
    return pl.pallas_call(
        _scales_kernel,
        grid=(GN,),
        in_specs=[pl.BlockSpec((NC, BN, 1), lambda i: (0, i, 0))],
        out_specs=[pl.BlockSpec((BN, 1), lambda i: (i, 0))] * 4,
        out_shape=[s, s, s, s],
    )(deg.reshape(NC, NP, 1))


def _init_user_kernel(x_ref, a_ref, out_ref):
    i = pl.program_id(0)
    rows = lax.broadcasted_iota(jnp.int32, (BN, 1), 0) + i * BN
    valid = rows < N
    h = jnp.where(valid, a_ref[...] * x_ref[...], 0.0)
    for c in range(NCH):
        out_ref[c] = h[:, c * CW:(c + 1) * CW]


def _init_user_call(user_emb, a):
    return pl.pallas_call(
        _init_user_kernel,
        grid=(GN,),
        in_specs=[
            pl.BlockSpec((BN, D), lambda i: (i, 0)),
            pl.BlockSpec((BN, 1), lambda i: (i, 0)),
        ],
        out_specs=pl.BlockSpec((NCH, BN, CW), lambda i: (0, i, 0)),
        out_shape=jax.ShapeDtypeStruct((NCH, NP, CW), jnp.float32),
    )(user_emb, a)


def _init_item_kernel(x_ref, w_ref, bias_ref, b_ref, out_ref):
    i = pl.program_id(0)
    rows = lax.broadcasted_iota(jnp.int32, (BN, 1), 0) + i * BN
    valid = rows < N
    h = lax.dot_general(x_ref[...], w_ref[...], (((1,), (1,)), ((), ())),
                        preferred_element_type=jnp.float32) + bias_ref[...]
    h = jnp.where(valid, b_ref[...] * h, 0.0)
    for c in range(NCH):
        out_ref[c] = h[:, c * CW:(c + 1) * CW]


def _init_item_call(item_emb, fc_w, fc_b, b):
    return pl.pallas_call(
        _init_item_kernel,
        grid=(GN,),
        in_specs=[
            pl.BlockSpec((BN, D), lambda i: (i, 0)),
            pl.BlockSpec((D, D), lambda i: (0, 0)),
            pl.BlockSpec((1, D), lambda i: (0, 0)),
            pl.BlockSpec((BN, 1), lambda i: (i, 0)),
        ],
        out_specs=pl.BlockSpec((NCH, BN, CW), lambda i: (0, i, 0)),
        out_shape=jax.ShapeDtypeStruct((NCH, NP, CW), jnp.float32),
    )(item_emb, fc_w, fc_b.reshape(1, D), b)


def _scale_kernel(s_ref, sc_ref, out_ref):
    i = pl.program_id(1)
    rows = lax.broadcasted_iota(jnp.int32, (1, BN, 1), 1) + i * BN
    valid = rows < N
    out_ref[...] = jnp.where(valid, sc_ref[...] * s_ref[...], 0.0)


def _scale_call(s, sc):
    return pl.pallas_call(
        _scale_kernel,
        grid=(NCH, GN),
        in_specs=[
            pl.BlockSpec((1, BN, CW), lambda c, i: (c, i, 0)),
            pl.BlockSpec((1, BN, 1), lambda c, i: (0, i, 0)),
        ],
        out_specs=pl.BlockSpec((1, BN, CW), lambda c, i: (c, i, 0)),
        out_shape=jax.ShapeDtypeStruct((NCH, NP, CW), jnp.float32),
    )(s, sc.reshape(1, NP, 1))


def _final_kernel(x_ref, sc_ref, s1_ref, s2_ref, out_ref):
    s1 = jnp.concatenate([s1_ref[c] for c in range(NCH)], axis=1)
    s2 = jnp.concatenate([s2_ref[c] for c in range(NCH)], axis=1)
    out_ref[...] = (x_ref[...] + sc_ref[...] * (s1 + s2)) * (1.0 / 3.0)


def _final_call(x, sc, s1, s2):
    return pl.pallas_call(
        _final_kernel,
        grid=(GN,),
        in_specs=[
            pl.BlockSpec((BN, D), lambda i: (i, 0)),
            pl.BlockSpec((BN, 1), lambda i: (i, 0)),
            pl.BlockSpec((NCH, BN, CW), lambda i: (0, i, 0)),
            pl.BlockSpec((NCH, BN, CW), lambda i: (0, i, 0)),
        ],
        out_specs=pl.BlockSpec((BN, D), lambda i: (i, 0)),
        out_shape=jax.ShapeDtypeStruct((N, D), jnp.float32),
    )(x, sc, s1, s2)


@jax.jit
def kernel(user_emb, item_emb, fc_w, fc_b, edge_src, edge_dst):
    pad = jnp.arange(EP - E, dtype=jnp.int32) % (NP - N) + N
    src_p = jnp.concatenate([edge_src, pad])
    dst_p = jnp.concatenate([edge_dst, pad])

    deg = _deg_call(src_p, dst_p)
    a, b, a2, b2 = _scales_call(deg)

    xu0 = _init_user_call(user_emb, a)
    xi0 = _init_item_call(item_emb, fc_w, fc_b, b)

    si1 = _seg_call(xu0.reshape(NCH * NP, CW), src_p, dst_p)
    su1 = _seg_call(xi0.reshape(NCH * NP, CW), dst_p, src_p)

    xi1 = _scale_call(si1.reshape(NCH, NP, CW), b2)
    xu1 = _scale_call(su1.reshape(NCH, NP, CW), a2)

    su2 = _seg_call(xi1.reshape(NCH * NP, CW), dst_p, src_p)
    si2 = _seg_call(xu1.reshape(NCH * NP, CW), src_p, dst_p)

    user_out = _final_call(user_emb, a, su1.reshape(NCH, NP, CW),
                           su2.reshape(NCH, NP, CW))
    item_out = _final_call(item_emb, b, si1.reshape(NCH, NP, CW),
                           si2.reshape(NCH, NP, CW))
    return user_out, item_out
